# baseline (speedup 1.0000x reference)
import functools

import jax
import jax.numpy as jnp
from jax import lax
from jax.experimental import pallas as pl
from jax.experimental.pallas import tpu as pltpu
from jax.experimental.pallas import tpu_sc as plsc

N = 10000
E = 320000
HP = 128
RP = 64
WPAIR = 256

NC = 2
NS = 16
NW = NC * NS
EPW = E // NW
C = 80
NCHUNK = EPW // C
RA = 624
ZR = 16
DR = 208

def _pad2(w, r, c):
    return jnp.zeros((r, c), jnp.float32).at[: w.shape[0], : w.shape[1]].set(w)


def _pad1(b, n):
    return jnp.zeros((1, n), jnp.float32).at[0, : b.shape[0]].set(b)


@functools.lru_cache(maxsize=None)
def _build_sc_msg():
    mesh = plsc.VectorSubcoreMesh(core_axis_name="c", subcore_axis_name="s")

    @functools.partial(
        pl.kernel,
        mesh=mesh,
        out_type=jax.ShapeDtypeStruct((NC, N, HP), jnp.float32),
        scratch_types=[
            pltpu.VMEM((2, C), jnp.int32),
            pltpu.VMEM((2, C), jnp.int32),
            pltpu.VMEM((2, C, HP), jnp.float32),
            pltpu.VMEM((2, C, HP), jnp.float32),
            pltpu.VMEM((ZR, HP), jnp.float32),
            pltpu.VMEM_SHARED((N, HP), jnp.float32),
            pltpu.SemaphoreType.DMA,
            pltpu.SemaphoreType.DMA,
        ],
    )
    def _sc_msg_kernel(h_hbm, elin_hbm, src_hbm, dst_hbm, out_hbm,
                       idx_s, idx_d, elin_v, rows_v, zbuf, acc, sem0, sem1):
        _sc_msg_body(h_hbm, elin_hbm, src_hbm, dst_hbm, out_hbm,
                     idx_s, idx_d, elin_v, rows_v, zbuf, acc, (sem0, sem1))

    return _sc_msg_kernel


def _sc_msg(h, elin, src, dst):
    return _build_sc_msg()(h, elin, src, dst)


def _sc_msg_body(h_hbm, elin_hbm, src_hbm, dst_hbm, out_hbm,
                 idx_s, idx_d, elin_v, rows_v, zbuf, acc, sems):
    cid = lax.axis_index("c")
    sid = lax.axis_index("s")
    base0 = (cid * NS + sid) * EPW

    def _zrow(r, _):
        for c8 in range(HP // 16):
            zbuf[r, pl.ds(c8 * 16, 16)] = jnp.zeros((16,), jnp.float32)
        return _
    lax.fori_loop(0, ZR, _zrow, None)
    row_base = sid * RA
    nz = jnp.where(sid == NS - 1, (RA + 16) // ZR, RA // ZR)

    def _zcp(j, _):
        pltpu.sync_copy(zbuf, acc.at[pl.ds(row_base + j * ZR, ZR)])
        return _
    lax.fori_loop(0, nz, _zcp, None)
    plsc.subcore_barrier()

    def _issue(k, b):
        base = base0 + k * C
        pltpu.sync_copy(src_hbm.at[pl.ds(base, C)], idx_s.at[b])
        pltpu.sync_copy(dst_hbm.at[pl.ds(base, C)], idx_d.at[b])
        pltpu.async_copy(elin_hbm.at[pl.ds(base, C)], elin_v.at[b], sems[b])
        pltpu.async_copy(h_hbm.at[idx_s.at[b]], rows_v.at[b], sems[b])

    def _proc(k, b):
        @pl.when(k < NCHUNK - 1)
        def _nxt():
            _issue(k + 1, 1 - b)
        base = base0 + k * C
        pltpu.make_async_copy(elin_hbm.at[pl.ds(base, C)], elin_v.at[b],
                              sems[b]).wait()
        pltpu.make_async_copy(h_hbm.at[idx_s.at[b]], rows_v.at[b],
                              sems[b]).wait()

        def _row(r, __):
            for dr in range(2):
                for c8 in range(HP // 16):
                    sl = pl.ds(c8 * 16, 16)
                    rows_v[b, 2 * r + dr, sl] = jnp.maximum(
                        rows_v[b, 2 * r + dr, sl] + elin_v[b, 2 * r + dr, sl], 0.0)
            return __
        lax.fori_loop(0, C // 2, _row, None)
        pltpu.sync_copy(rows_v.at[b], acc.at[idx_d.at[b]], add=True)

    _issue(0, 0)

    def _chunk(k, _):
        @pl.when(k % 2 == 0)
        def _even():
            _proc(k, 0)

        @pl.when(k % 2 == 1)
        def _odd():
            _proc(k, 1)
        return _
    lax.fori_loop(0, NCHUNK, _chunk, None)

    plsc.subcore_barrier()
    for j in range(RA // DR):
        row0 = row_base + j * DR
        pltpu.sync_copy(acc.at[pl.ds(row0, DR)], out_hbm.at[cid, pl.ds(row0, DR)])

    @pl.when(sid == NS - 1)
    def _dtail():
        pltpu.sync_copy(acc.at[pl.ds(row_base + RA, 16)],
                        out_hbm.at[cid, pl.ds(row_base + RA, 16)])


@functools.lru_cache(maxsize=None)
def _make_sc_pair(width):
    mesh = plsc.VectorSubcoreMesh(core_axis_name="c", subcore_axis_name="s")

    @functools.partial(
        pl.kernel,
        mesh=mesh,
        out_type=jax.ShapeDtypeStruct((E, width), jnp.float32),
        scratch_types=[
            pltpu.VMEM((2, C), jnp.int32),
            pltpu.VMEM((2, C), jnp.int32),
            pltpu.VMEM((2, C, width), jnp.float32),
            pltpu.VMEM((2, C, width), jnp.float32),
            pltpu.SemaphoreType.DMA,
            pltpu.SemaphoreType.DMA,
        ],
    )
    def _sc_pair(ts_hbm, td_hbm, src_hbm, dst_hbm, out_hbm,
                 idx_s, idx_d, gs, gd, sem0, sem1):
        cid = lax.axis_index("c")
        sid = lax.axis_index("s")
        base0 = (cid * NS + sid) * EPW
        sems = (sem0, sem1)

        def _issue(k, b):
            base = base0 + k * C
            pltpu.sync_copy(src_hbm.at[pl.ds(base, C)], idx_s.at[b])
            pltpu.sync_copy(dst_hbm.at[pl.ds(base, C)], idx_d.at[b])
            pltpu.async_copy(ts_hbm.at[idx_s.at[b]], gs.at[b], sems[b])
            pltpu.async_copy(td_hbm.at[idx_d.at[b]], gd.at[b], sems[b])

        def _proc(k, b):
            @pl.when(k < NCHUNK - 1)
            def _nxt():
                _issue(k + 1, 1 - b)
            pltpu.make_async_copy(ts_hbm.at[idx_s.at[b]], gs.at[b],
                                  sems[b]).wait()
            pltpu.make_async_copy(td_hbm.at[idx_d.at[b]], gd.at[b],
                                  sems[b]).wait()

            def _row(r, __):
                for dr in range(2):
                    for c8 in range(width // 16):
                        sl = pl.ds(c8 * 16, 16)
                        gs[b, 2 * r + dr, sl] = (gs[b, 2 * r + dr, sl]
                                                 + gd[b, 2 * r + dr, sl])
                return __
            lax.fori_loop(0, C // 2, _row, None)
            base = base0 + k * C
            pltpu.sync_copy(gs.at[b], out_hbm.at[pl.ds(base, C)])

        _issue(0, 0)

        def _chunk(k, _):
            @pl.when(k % 2 == 0)
            def _even():
                _proc(k, 0)

            @pl.when(k % 2 == 1)
            def _odd():
                _proc(k, 1)
            return _
        lax.fori_loop(0, NCHUNK, _chunk, None)

    return _sc_pair


def _sc_pair_h(ts, td, src, dst):
    return _make_sc_pair(HP)(ts, td, src, dst)


def _sc_pair_w(ts, td, src, dst):
    return _make_sc_pair(WPAIR)(ts, td, src, dst)


_EB = 2000


def _dot(a, b):
    return jnp.dot(a, b, preferred_element_type=jnp.float32)


def _bdot(a, b):
    return jnp.dot(a.astype(jnp.bfloat16), b.astype(jnp.bfloat16),
                   preferred_element_type=jnp.float32)


def _node_emb_body(x_ref, w_ref, b_ref, h_ref):
    h_ref[...] = _dot(x_ref[...], w_ref[...]) + b_ref[...]


def _edge0_body(ea_ref, we_ref, be_ref, gw_ref, gb_ref, e0_ref, el_ref):
    e0 = _bdot(ea_ref[...], we_ref[...]) + be_ref[...]
    e0_ref[...] = e0
    el_ref[...] = _bdot(e0, gw_ref[...]) + gb_ref[...]


def _node_upd0_body(h_ref, ag_ref, w1_ref, b1_ref, w2_ref, b2_ref,
                    gam_ref, bet_ref, w1a_ref, w1b_ref,
                    hn_ref, ts_ref, td_ref):
    h = h_ref[...]
    g = h + ag_ref[0] + ag_ref[1]
    t = jnp.maximum(_dot(g, w1_ref[...]) + b1_ref[...], 0.0)
    nn = _dot(t, w2_ref[...]) + b2_ref[...]
    mu = jnp.mean(nn, axis=0, keepdims=True)
    var = jnp.mean((nn - mu) * (nn - mu), axis=0, keepdims=True)
    bn = (nn - mu) * lax.rsqrt(var + 1e-5) * gam_ref[...] + bet_ref[...]
    hn = (h + jnp.maximum(bn, 0.0)) * 0.5
    hn_ref[...] = hn
    ts_ref[...] = _dot(hn, w1a_ref[...])
    td_ref[...] = _dot(hn, w1b_ref[...])


def _node_upd1_body(h_ref, ag_ref, w1_ref, b1_ref, w2_ref, b2_ref,
                    gam_ref, bet_ref, w1a_ref, w1b_ref, mwa_ref, mwb_ref,
                    hn_ref, ts_ref, td_ref):
    h = h_ref[...]
    g = h + ag_ref[0] + ag_ref[1]
    t = jnp.maximum(_dot(g, w1_ref[...]) + b1_ref[...], 0.0)
    nn = _dot(t, w2_ref[...]) + b2_ref[...]
    mu = jnp.mean(nn, axis=0, keepdims=True)
    var = jnp.mean((nn - mu) * (nn - mu), axis=0, keepdims=True)
    bn = (nn - mu) * lax.rsqrt(var + 1e-5) * gam_ref[...] + bet_ref[...]
    hn = (h + jnp.maximum(bn, 0.0)) * 0.5
    hn_ref[...] = hn
    hr = jnp.maximum(hn, 0.0)
    z = jnp.zeros((hn.shape[0], WPAIR - HP - RP), jnp.float32)
    ts_ref[...] = jnp.concatenate(
        [_dot(hn, w1a_ref[...]), _dot(hr, mwa_ref[...]), z], axis=1)
    td_ref[...] = jnp.concatenate(
        [_dot(hn, w1b_ref[...]), _dot(hr, mwb_ref[...]), z], axis=1)


def _edge_upd_body(s_ref, e_ref, w1c_ref, eb1_ref, ew2_ref, eb2_ref,
                   gw_ref, gb_ref, e1_ref, el_ref):
    e = e_ref[...]
    t = jnp.maximum(s_ref[...] + _bdot(e, w1c_ref[...]) + eb1_ref[...], 0.0)
    em = _bdot(t, ew2_ref[...]) + eb2_ref[...]
    e1 = e + em * 0.5
    e1_ref[...] = e1
    el_ref[...] = _bdot(e1, gw_ref[...]) + gb_ref[...]


def _edge_final_body(s_ref, e_ref, w1c_ref, eb1_ref, ew2_ref, eb2_ref,
                     mwc_ref, mb1_ref, mw2_ref, mb2_ref, mw3_ref, mb3_ref,
                     out_ref):
    e = e_ref[...]
    t = jnp.maximum(s_ref[:, :HP] + _bdot(e, w1c_ref[...]) + eb1_ref[...], 0.0)
    em = _bdot(t, ew2_ref[...]) + eb2_ref[...]
    e2 = e + em * 0.5
    z1 = jnp.maximum(s_ref[:, HP:HP + RP] + _bdot(e2, mwc_ref[...]) + mb1_ref[...], 0.0)
    z2 = jnp.maximum(_bdot(z1, mw2_ref[...]) + mb2_ref[...], 0.0)
    out_ref[...] = _bdot(z2, mw3_ref[...]) + mb3_ref[...]


def _wspec(r, c):
    return pl.BlockSpec((r, c), lambda i: (0, 0))


def _f32(shape):
    return jax.ShapeDtypeStruct(shape, jnp.float32)


def _node_emb(x, wn, bn):
    return pl.pallas_call(_node_emb_body, out_shape=_f32((N, HP)))(x, wn, bn)


def _edge0(ea, we, be, gw, gb):
    grid = E // _EB
    return pl.pallas_call(
        _edge0_body,
        grid=grid,
        in_specs=[pl.BlockSpec((_EB, 16), lambda i: (i, 0)),
                  _wspec(16, HP), _wspec(1, HP), _wspec(HP, HP), _wspec(1, HP)],
        out_specs=[pl.BlockSpec((_EB, HP), lambda i: (i, 0)),
                   pl.BlockSpec((_EB, HP), lambda i: (i, 0))],
        out_shape=[_f32((E, HP)), _f32((E, HP))],
    )(ea, we, be, gw, gb)


def _node_upd0(h, ag, w):
    return pl.pallas_call(
        _node_upd0_body,
        out_shape=[_f32((N, HP)), _f32((N, HP)), _f32((N, HP))],
    )(h, ag, w['W1'], w['b1'], w['W2'], w['b2'], w['gam'], w['bet'],
      w['W1a'], w['W1b'])


def _node_upd1(h, ag, w, mwa, mwb):
    return pl.pallas_call(
        _node_upd1_body,
        out_shape=[_f32((N, HP)), _f32((N, WPAIR)), _f32((N, WPAIR))],
    )(h, ag, w['W1'], w['b1'], w['W2'], w['b2'], w['gam'], w['bet'],
      w['W1a'], w['W1b'], mwa, mwb)


def _edge_upd(s, e, w, gw, gb):
    grid = E // _EB
    eb = pl.BlockSpec((_EB, HP), lambda i: (i, 0))
    return pl.pallas_call(
        _edge_upd_body,
        grid=grid,
        in_specs=[eb, eb, _wspec(HP, HP), _wspec(1, HP), _wspec(HP, HP),
                  _wspec(1, HP), _wspec(HP, HP), _wspec(1, HP)],
        out_specs=[eb, eb],
        out_shape=[_f32((E, HP)), _f32((E, HP))],
    )(s, e, w['W1c'], w['eb1'], w['eW2'], w['eb2'], gw, gb)


def _edge_final(s, e, w, mwc, mb1, mw2, mb2, mw3, mb3):
    grid = E // _EB
    return pl.pallas_call(
        _edge_final_body,
        grid=grid,
        in_specs=[pl.BlockSpec((_EB, WPAIR), lambda i: (i, 0)),
                  pl.BlockSpec((_EB, HP), lambda i: (i, 0)),
                  _wspec(HP, HP), _wspec(1, HP), _wspec(HP, HP), _wspec(1, HP),
                  _wspec(HP, RP), _wspec(1, RP), _wspec(RP, RP), _wspec(1, RP),
                  _wspec(RP, 2), _wspec(1, 2)],
        out_specs=pl.BlockSpec((_EB, 2), lambda i: (i, 0)),
        out_shape=_f32((E, 2)),
    )(s, e, w['W1c'], w['eb1'], w['eW2'], w['eb2'],
      mwc, mb1, mw2, mb2, mw3, mb3)


def _layer_weights(p):
    return dict(
        gW=_pad2(p['gine_lin_W'], HP, HP), gb=_pad1(p['gine_lin_b'], HP),
        W1=_pad2(p['gmlp_W1'], HP, HP), b1=_pad1(p['gmlp_b1'], HP),
        W2=_pad2(p['gmlp_W2'], HP, HP), b2=_pad1(p['gmlp_b2'], HP),
        gam=_pad1(p['bn_gamma'], HP), bet=_pad1(p['bn_beta'], HP),
        W1a=_pad2(p['emlp_W1'][0:100], HP, HP),
        W1b=_pad2(p['emlp_W1'][100:200], HP, HP),
        W1c=_pad2(p['emlp_W1'][200:300], HP, HP),
        eb1=_pad1(p['emlp_b1'], HP),
        eW2=_pad2(p['emlp_W2'], HP, HP), eb2=_pad1(p['emlp_b2'], HP),
    )


def kernel(x, edge_attr, params, edge_index):
    src = edge_index[0]
    dst = edge_index[1]
    P = params
    w0 = _layer_weights(P['layers'][0])
    w1 = _layer_weights(P['layers'][1])
    wn = _pad2(P['node_emb_W'], 128, HP)
    bn = _pad1(P['node_emb_b'], HP)
    we = _pad2(P['edge_emb_W'], 16, HP)
    be = _pad1(P['edge_emb_b'], HP)
    mwa = _pad2(P['mlp_W1'][0:100], HP, RP)
    mwb = _pad2(P['mlp_W1'][100:200], HP, RP)
    mwc = _pad2(P['mlp_W1'][200:300], HP, RP)
    mb1 = _pad1(P['mlp_b1'], RP)
    mw2 = _pad2(P['mlp_W2'], RP, RP)
    mb2 = _pad1(P['mlp_b2'], RP)
    mw3 = _pad2(P['mlp_W3'], RP, 2)
    mb3 = _pad1(P['mlp_b3'], 2)

    h = _node_emb(x, wn, bn)
    e, elin = _edge0(edge_attr, we, be, w0['gW'], w0['gb'])

    ag = _sc_msg(h, elin, src, dst)
    h, ts, td = _node_upd0(h, ag, w0)
    s = _sc_pair_h(ts, td, src, dst)
    e, elin = _edge_upd(s, e, w0, w1['gW'], w1['gb'])

    ag = _sc_msg(h, elin, src, dst)
    h, ts, td = _node_upd1(h, ag, w1, mwa, mwb)
    s = _sc_pair_w(ts, td, src, dst)
    out = _edge_final(s, e, w1, mwc, mb1, mw2, mb2, mw3, mb3)
    return out


# speedup vs baseline: 1.1314x; 1.1314x over previous
import functools

import jax
import jax.numpy as jnp
from jax import lax
from jax.experimental import pallas as pl
from jax.experimental.pallas import tpu as pltpu
from jax.experimental.pallas import tpu_sc as plsc

N = 10000
E = 320000
HP = 128
RP = 64
WPAIR = 256

NC = 2
NS = 16
NW = NC * NS
EPW = E // NW
C = 80
NCHUNK = EPW // C
RA = 624
ZR = 16
DR = 208

_MASK_HI = -65536


def _bits(x):
    return jax.lax.bitcast_convert_type(x, jnp.int32)


def _f32cast(w):
    return jax.lax.bitcast_convert_type(w, jnp.float32)


def _rb16(x):
    return x.astype(jnp.bfloat16).astype(jnp.float32)


def _pack_cols(a, b):
    return (_bits(_rb16(a)) & _MASK_HI) | jax.lax.shift_right_logical(
        _bits(_rb16(b)), 16)


def _unpack_hi(w):
    return _f32cast(w & _MASK_HI)


def _unpack_lo(w):
    return _f32cast(jax.lax.shift_left(w, 16))


def _pad2(w, r, c):
    return jnp.zeros((r, c), jnp.float32).at[: w.shape[0], : w.shape[1]].set(w)


def _pad1(b, n):
    return jnp.zeros((1, n), jnp.float32).at[0, : b.shape[0]].set(b)


@functools.lru_cache(maxsize=None)
def _build_sc_msg():
    mesh = plsc.VectorSubcoreMesh(core_axis_name="c", subcore_axis_name="s")

    @functools.partial(
        pl.kernel,
        mesh=mesh,
        out_type=jax.ShapeDtypeStruct((NC, N, HP), jnp.float32),
        scratch_types=[
            pltpu.VMEM((2, C), jnp.int32),
            pltpu.VMEM((2, C), jnp.int32),
            pltpu.VMEM((2, C, HP), jnp.float32),
            pltpu.VMEM((2, C, HP), jnp.float32),
            pltpu.VMEM((ZR, HP), jnp.float32),
            pltpu.VMEM_SHARED((N, HP), jnp.float32),
            pltpu.SemaphoreType.DMA,
            pltpu.SemaphoreType.DMA,
        ],
    )
    def _sc_msg_kernel(h_hbm, elin_hbm, src_hbm, dst_hbm, out_hbm,
                       idx_s, idx_d, elin_v, rows_v, zbuf, acc, sem0, sem1):
        _sc_msg_body(h_hbm, elin_hbm, src_hbm, dst_hbm, out_hbm,
                     idx_s, idx_d, elin_v, rows_v, zbuf, acc, (sem0, sem1))

    return _sc_msg_kernel


def _sc_msg(h, elin, src, dst):
    return _build_sc_msg()(h, elin, src, dst)


def _sc_msg_body(h_hbm, elin_hbm, src_hbm, dst_hbm, out_hbm,
                 idx_s, idx_d, elin_v, rows_v, zbuf, acc, sems):
    cid = lax.axis_index("c")
    sid = lax.axis_index("s")
    base0 = (cid * NS + sid) * EPW

    def _zrow(r, _):
        for c8 in range(HP // 16):
            zbuf[r, pl.ds(c8 * 16, 16)] = jnp.zeros((16,), jnp.float32)
        return _
    lax.fori_loop(0, ZR, _zrow, None)
    row_base = sid * RA
    nz = jnp.where(sid == NS - 1, (RA + 16) // ZR, RA // ZR)

    def _zcp(j, _):
        pltpu.sync_copy(zbuf, acc.at[pl.ds(row_base + j * ZR, ZR)])
        return _
    lax.fori_loop(0, nz, _zcp, None)
    plsc.subcore_barrier()

    def _issue(k, b):
        base = base0 + k * C
        pltpu.sync_copy(src_hbm.at[pl.ds(base, C)], idx_s.at[b])
        pltpu.sync_copy(dst_hbm.at[pl.ds(base, C)], idx_d.at[b])
        pltpu.async_copy(elin_hbm.at[pl.ds(base, C)], elin_v.at[b], sems[b])
        pltpu.async_copy(h_hbm.at[idx_s.at[b]], rows_v.at[b], sems[b])

    def _proc(k, b):
        @pl.when(k < NCHUNK - 1)
        def _nxt():
            _issue(k + 1, 1 - b)
        base = base0 + k * C
        pltpu.make_async_copy(elin_hbm.at[pl.ds(base, C)], elin_v.at[b],
                              sems[b]).wait()
        pltpu.make_async_copy(h_hbm.at[idx_s.at[b]], rows_v.at[b],
                              sems[b]).wait()

        def _row(r, __):
            for dr in range(2):
                row = 2 * r + dr
                for c8 in range(HP // 16):
                    sl = pl.ds(c8 * 16, 16)
                    rows_v[b, row, sl] = jnp.maximum(
                        rows_v[b, row, sl] + elin_v[b, row, sl], 0.0)
            return __
        lax.fori_loop(0, C // 2, _row, None)
        pltpu.sync_copy(rows_v.at[b], acc.at[idx_d.at[b]], add=True)

    _issue(0, 0)

    def _chunk(k, _):
        @pl.when(k % 2 == 0)
        def _even():
            _proc(k, 0)

        @pl.when(k % 2 == 1)
        def _odd():
            _proc(k, 1)
        return _
    lax.fori_loop(0, NCHUNK, _chunk, None)

    plsc.subcore_barrier()
    for j in range(RA // DR):
        row0 = row_base + j * DR
        pltpu.sync_copy(acc.at[pl.ds(row0, DR)], out_hbm.at[cid, pl.ds(row0, DR)])

    @pl.when(sid == NS - 1)
    def _dtail():
        pltpu.sync_copy(acc.at[pl.ds(row_base + RA, 16)],
                        out_hbm.at[cid, pl.ds(row_base + RA, 16)])


def _pair_pipeline(ts_hbm, td_hbm, src_hbm, dst_hbm, out_hbm,
                   idx_s, idx_d, gs, gd, sems, compute_store):
    cid = lax.axis_index("c")
    sid = lax.axis_index("s")
    base0 = (cid * NS + sid) * EPW

    def _issue(k, b):
        base = base0 + k * C
        pltpu.sync_copy(src_hbm.at[pl.ds(base, C)], idx_s.at[b])
        pltpu.sync_copy(dst_hbm.at[pl.ds(base, C)], idx_d.at[b])
        pltpu.async_copy(ts_hbm.at[idx_s.at[b]], gs.at[b], sems[b])
        pltpu.async_copy(td_hbm.at[idx_d.at[b]], gd.at[b], sems[b])

    def _proc(k, b):
        @pl.when(k < NCHUNK - 1)
        def _nxt():
            _issue(k + 1, 1 - b)
        pltpu.make_async_copy(ts_hbm.at[idx_s.at[b]], gs.at[b], sems[b]).wait()
        pltpu.make_async_copy(td_hbm.at[idx_d.at[b]], gd.at[b], sems[b]).wait()
        compute_store(k, b, base0)

    _issue(0, 0)

    def _chunk(k, _):
        @pl.when(k % 2 == 0)
        def _even():
            _proc(k, 0)

        @pl.when(k % 2 == 1)
        def _odd():
            _proc(k, 1)
        return _
    lax.fori_loop(0, NCHUNK, _chunk, None)


@functools.lru_cache(maxsize=None)
def _build_sc_pair_f32():
    mesh = plsc.VectorSubcoreMesh(core_axis_name="c", subcore_axis_name="s")

    @functools.partial(
        pl.kernel,
        mesh=mesh,
        out_type=jax.ShapeDtypeStruct((E, HP), jnp.float32),
        scratch_types=[
            pltpu.VMEM((2, C), jnp.int32),
            pltpu.VMEM((2, C), jnp.int32),
            pltpu.VMEM((2, C, HP), jnp.float32),
            pltpu.VMEM((2, C, HP), jnp.float32),
            pltpu.SemaphoreType.DMA,
            pltpu.SemaphoreType.DMA,
        ],
    )
    def _sc_pair(ts_hbm, td_hbm, src_hbm, dst_hbm, out_hbm,
                 idx_s, idx_d, gs, gd, sem0, sem1):
        def _cs(k, b, base0):
            def _row(r, __):
                for dr in range(2):
                    row = 2 * r + dr
                    for c8 in range(HP // 16):
                        sl = pl.ds(c8 * 16, 16)
                        gs[b, row, sl] = gs[b, row, sl] + gd[b, row, sl]
                return __
            lax.fori_loop(0, C // 2, _row, None)
            pltpu.sync_copy(gs.at[b], out_hbm.at[pl.ds(base0 + k * C, C)])

        _pair_pipeline(ts_hbm, td_hbm, src_hbm, dst_hbm, out_hbm,
                       idx_s, idx_d, gs, gd, (sem0, sem1), _cs)

    return _sc_pair


@functools.lru_cache(maxsize=None)
def _build_sc_pair_packed():
    mesh = plsc.VectorSubcoreMesh(core_axis_name="c", subcore_axis_name="s")

    @functools.partial(
        pl.kernel,
        mesh=mesh,
        out_type=[jax.ShapeDtypeStruct((E, HP), jnp.int32),
                  jax.ShapeDtypeStruct((E, HP), jnp.int32)],
        scratch_types=[
            pltpu.VMEM((2, C), jnp.int32),
            pltpu.VMEM((2, C), jnp.int32),
            pltpu.VMEM((2, C, HP), jnp.int32),
            pltpu.VMEM((2, C, HP), jnp.int32),
            pltpu.SemaphoreType.DMA,
            pltpu.SemaphoreType.DMA,
        ],
    )
    def _sc_pair(ts_hbm, td_hbm, src_hbm, dst_hbm, outs_hbm, outd_hbm,
                 idx_s, idx_d, gs, gd, sem0, sem1):
        def _cs(k, b, base0):
            base = base0 + k * C
            pltpu.sync_copy(gs.at[b], outs_hbm.at[pl.ds(base, C)])
            pltpu.sync_copy(gd.at[b], outd_hbm.at[pl.ds(base, C)])

        _pair_pipeline(ts_hbm, td_hbm, src_hbm, dst_hbm, None,
                       idx_s, idx_d, gs, gd, (sem0, sem1), _cs)

    return _sc_pair


def _sc_pair_h(ts, td, src, dst):
    return _build_sc_pair_f32()(ts, td, src, dst)


def _sc_pair_w(ts, td, src, dst):
    return _build_sc_pair_packed()(ts, td, src, dst)


_EB = 2000


def _dot(a, b):
    return jnp.dot(a, b, preferred_element_type=jnp.float32)


def _node_emb_body(x_ref, w_ref, b_ref, h_ref):
    h_ref[...] = _dot(x_ref[...], w_ref[...]) + b_ref[...]


def _edge0_body(ea_ref, we_ref, be_ref, gw_ref, gb_ref, e0_ref, el_ref):
    e0 = _dot(ea_ref[...], we_ref[...]) + be_ref[...]
    e0_ref[...] = e0.astype(jnp.bfloat16)
    el_ref[...] = _dot(e0, gw_ref[...]) + gb_ref[...]


def _node_upd0_body(h_ref, ag_ref, w1_ref, b1_ref, w2_ref, b2_ref,
                    gam_ref, bet_ref, w1a_ref, w1b_ref,
                    hn_ref, ts_ref, td_ref):
    h = h_ref[...]
    g = h + ag_ref[0] + ag_ref[1]
    t = jnp.maximum(_dot(g, w1_ref[...]) + b1_ref[...], 0.0)
    nn = _dot(t, w2_ref[...]) + b2_ref[...]
    mu = jnp.mean(nn, axis=0, keepdims=True)
    var = jnp.mean((nn - mu) * (nn - mu), axis=0, keepdims=True)
    bn = (nn - mu) * lax.rsqrt(var + 1e-5) * gam_ref[...] + bet_ref[...]
    hn = (h + jnp.maximum(bn, 0.0)) * 0.5
    hn_ref[...] = hn
    ts_ref[...] = _dot(hn, w1a_ref[...])
    td_ref[...] = _dot(hn, w1b_ref[...])


def _node_upd1_body(h_ref, ag_ref, w1_ref, b1_ref, w2_ref, b2_ref,
                    gam_ref, bet_ref, w1a_ref, w1b_ref, mwa_ref, mwb_ref,
                    hn_ref, ts_ref, td_ref):
    h = h_ref[...]
    g = h + ag_ref[0] + ag_ref[1]
    t = jnp.maximum(_dot(g, w1_ref[...]) + b1_ref[...], 0.0)
    nn = _dot(t, w2_ref[...]) + b2_ref[...]
    mu = jnp.mean(nn, axis=0, keepdims=True)
    var = jnp.mean((nn - mu) * (nn - mu), axis=0, keepdims=True)
    bn = (nn - mu) * lax.rsqrt(var + 1e-5) * gam_ref[...] + bet_ref[...]
    hn = (h + jnp.maximum(bn, 0.0)) * 0.5
    hn_ref[...] = hn
    hr = jnp.maximum(hn, 0.0)
    z = jnp.zeros((hn.shape[0], 32), jnp.int32)
    a = _dot(hn, w1a_ref[...])
    r = _dot(hr, mwa_ref[...])
    ts_ref[...] = jnp.concatenate(
        [_pack_cols(a[:, :64], a[:, 64:]),
         _pack_cols(r[:, :32], r[:, 32:64]), z], axis=1)
    a = _dot(hn, w1b_ref[...])
    r = _dot(hr, mwb_ref[...])
    td_ref[...] = jnp.concatenate(
        [_pack_cols(a[:, :64], a[:, 64:]),
         _pack_cols(r[:, :32], r[:, 32:64]), z], axis=1)


def _edge_upd_body(s_ref, e_ref, w1c_ref, eb1_ref, ew2_ref, eb2_ref,
                   gw_ref, gb_ref, e1_ref, el_ref):
    e = e_ref[...].astype(jnp.float32)
    t = jnp.maximum(s_ref[...] + _dot(e, w1c_ref[...]) + eb1_ref[...], 0.0)
    em = _dot(t, ew2_ref[...]) + eb2_ref[...]
    e1 = e + em * 0.5
    e1_ref[...] = e1.astype(jnp.bfloat16)
    el_ref[...] = _dot(e1, gw_ref[...]) + gb_ref[...]


def _edge_final_body(ss_ref, sd_ref, e_ref, w1c_ref, eb1_ref, ew2_ref, eb2_ref,
                     mwc_ref, mb1_ref, mw2_ref, mb2_ref, mw3_ref, mb3_ref,
                     out_ref):
    e = e_ref[...].astype(jnp.float32)
    ws = ss_ref[...]
    wd = sd_ref[...]
    sa = jnp.concatenate(
        [_unpack_hi(ws[:, :64]) + _unpack_hi(wd[:, :64]),
         _unpack_lo(ws[:, :64]) + _unpack_lo(wd[:, :64])], axis=1)
    sr = jnp.concatenate(
        [_unpack_hi(ws[:, 64:96]) + _unpack_hi(wd[:, 64:96]),
         _unpack_lo(ws[:, 64:96]) + _unpack_lo(wd[:, 64:96])], axis=1)
    t = jnp.maximum(sa + _dot(e, w1c_ref[...]) + eb1_ref[...], 0.0)
    em = _dot(t, ew2_ref[...]) + eb2_ref[...]
    e2 = e + em * 0.5
    z1 = jnp.maximum(sr + _dot(e2, mwc_ref[...]) + mb1_ref[...], 0.0)
    z2 = jnp.maximum(_dot(z1, mw2_ref[...]) + mb2_ref[...], 0.0)
    out_ref[...] = _dot(z2, mw3_ref[...]) + mb3_ref[...]


def _wspec(r, c):
    return pl.BlockSpec((r, c), lambda i: (0, 0))


def _f32(shape):
    return jax.ShapeDtypeStruct(shape, jnp.float32)


def _node_emb(x, wn, bn):
    return pl.pallas_call(_node_emb_body, out_shape=_f32((N, HP)))(x, wn, bn)


def _edge0(ea, we, be, gw, gb):
    grid = E // _EB
    return pl.pallas_call(
        _edge0_body,
        grid=grid,
        in_specs=[pl.BlockSpec((_EB, 16), lambda i: (i, 0)),
                  _wspec(16, HP), _wspec(1, HP), _wspec(HP, HP), _wspec(1, HP)],
        out_specs=[pl.BlockSpec((_EB, HP), lambda i: (i, 0)),
                   pl.BlockSpec((_EB, HP), lambda i: (i, 0))],
        out_shape=[jax.ShapeDtypeStruct((E, HP), jnp.bfloat16),
                   _f32((E, HP))],
    )(ea, we, be, gw, gb)


def _node_upd0(h, ag, w):
    return pl.pallas_call(
        _node_upd0_body,
        out_shape=[_f32((N, HP)), _f32((N, HP)), _f32((N, HP))],
    )(h, ag, w['W1'], w['b1'], w['W2'], w['b2'], w['gam'], w['bet'],
      w['W1a'], w['W1b'])


def _node_upd1(h, ag, w, mwa, mwb):
    return pl.pallas_call(
        _node_upd1_body,
        out_shape=[_f32((N, HP)),
                   jax.ShapeDtypeStruct((N, HP), jnp.int32),
                   jax.ShapeDtypeStruct((N, HP), jnp.int32)],
    )(h, ag, w['W1'], w['b1'], w['W2'], w['b2'], w['gam'], w['bet'],
      w['W1a'], w['W1b'], mwa, mwb)


def _edge_upd(s, e, w, gw, gb):
    grid = E // _EB
    eb = pl.BlockSpec((_EB, HP), lambda i: (i, 0))
    return pl.pallas_call(
        _edge_upd_body,
        grid=grid,
        in_specs=[eb, eb, _wspec(HP, HP), _wspec(1, HP), _wspec(HP, HP),
                  _wspec(1, HP), _wspec(HP, HP), _wspec(1, HP)],
        out_specs=[eb, eb],
        out_shape=[jax.ShapeDtypeStruct((E, HP), jnp.bfloat16),
                   _f32((E, HP))],
    )(s, e, w['W1c'], w['eb1'], w['eW2'], w['eb2'], gw, gb)


def _edge_final(ss, sd, e, w, mwc, mb1, mw2, mb2, mw3, mb3):
    grid = E // _EB
    return pl.pallas_call(
        _edge_final_body,
        grid=grid,
        in_specs=[pl.BlockSpec((_EB, HP), lambda i: (i, 0)),
                  pl.BlockSpec((_EB, HP), lambda i: (i, 0)),
                  pl.BlockSpec((_EB, HP), lambda i: (i, 0)),
                  _wspec(HP, HP), _wspec(1, HP), _wspec(HP, HP), _wspec(1, HP),
                  _wspec(HP, RP), _wspec(1, RP), _wspec(RP, RP), _wspec(1, RP),
                  _wspec(RP, 2), _wspec(1, 2)],
        out_specs=pl.BlockSpec((_EB, 2), lambda i: (i, 0)),
        out_shape=_f32((E, 2)),
    )(ss, sd, e, w['W1c'], w['eb1'], w['eW2'], w['eb2'],
      mwc, mb1, mw2, mb2, mw3, mb3)


def _layer_weights(p):
    return dict(
        gW=_pad2(p['gine_lin_W'], HP, HP), gb=_pad1(p['gine_lin_b'], HP),
        W1=_pad2(p['gmlp_W1'], HP, HP), b1=_pad1(p['gmlp_b1'], HP),
        W2=_pad2(p['gmlp_W2'], HP, HP), b2=_pad1(p['gmlp_b2'], HP),
        gam=_pad1(p['bn_gamma'], HP), bet=_pad1(p['bn_beta'], HP),
        W1a=_pad2(p['emlp_W1'][0:100], HP, HP),
        W1b=_pad2(p['emlp_W1'][100:200], HP, HP),
        W1c=_pad2(p['emlp_W1'][200:300], HP, HP),
        eb1=_pad1(p['emlp_b1'], HP),
        eW2=_pad2(p['emlp_W2'], HP, HP), eb2=_pad1(p['emlp_b2'], HP),
    )


def kernel(x, edge_attr, params, edge_index):
    src = edge_index[0]
    dst = edge_index[1]
    P = params
    w0 = _layer_weights(P['layers'][0])
    w1 = _layer_weights(P['layers'][1])
    wn = _pad2(P['node_emb_W'], 128, HP)
    bn = _pad1(P['node_emb_b'], HP)
    we = _pad2(P['edge_emb_W'], 16, HP)
    be = _pad1(P['edge_emb_b'], HP)
    mwa = _pad2(P['mlp_W1'][0:100], HP, RP)
    mwb = _pad2(P['mlp_W1'][100:200], HP, RP)
    mwc = _pad2(P['mlp_W1'][200:300], HP, RP)
    mb1 = _pad1(P['mlp_b1'], RP)
    mw2 = _pad2(P['mlp_W2'], RP, RP)
    mb2 = _pad1(P['mlp_b2'], RP)
    mw3 = _pad2(P['mlp_W3'], RP, 2)
    mb3 = _pad1(P['mlp_b3'], 2)

    h = _node_emb(x, wn, bn)
    e, elin = _edge0(edge_attr, we, be, w0['gW'], w0['gb'])

    ag = _sc_msg(h, elin, src, dst)
    h, ts, td = _node_upd0(h, ag, w0)
    s = _sc_pair_h(ts, td, src, dst)
    e, elin = _edge_upd(s, e, w0, w1['gW'], w1['gb'])

    ag = _sc_msg(h, elin, src, dst)
    h, ts, td = _node_upd1(h, ag, w1, mwa, mwb)
    ss, sd = _sc_pair_w(ts, td, src, dst)
    out = _edge_final(ss, sd, e, w1, mwc, mb1, mw2, mb2, mw3, mb3)
    return out


# speedup vs baseline: 1.1318x; 1.0003x over previous
import functools

import jax
import jax.numpy as jnp
from jax import lax
from jax.experimental import pallas as pl
from jax.experimental.pallas import tpu as pltpu
from jax.experimental.pallas import tpu_sc as plsc

N = 10000
E = 320000
HP = 128
RP = 64
WPAIR = 256

NC = 2
NS = 16
NW = NC * NS
EPW = E // NW
C = 80
NCHUNK = EPW // C
RA = 624
ZR = 16
DR = 208

_MASK_HI = -65536


def _bits(x):
    return jax.lax.bitcast_convert_type(x, jnp.int32)


def _f32cast(w):
    return jax.lax.bitcast_convert_type(w, jnp.float32)


def _rb16(x):
    return x.astype(jnp.bfloat16).astype(jnp.float32)


def _pack_cols(a, b):
    return (_bits(_rb16(a)) & _MASK_HI) | jax.lax.shift_right_logical(
        _bits(_rb16(b)), 16)


def _unpack_hi(w):
    return _f32cast(w & _MASK_HI)


def _unpack_lo(w):
    return _f32cast(jax.lax.shift_left(w, 16))


def _pad2(w, r, c):
    return jnp.zeros((r, c), jnp.float32).at[: w.shape[0], : w.shape[1]].set(w)


def _pad1(b, n):
    return jnp.zeros((1, n), jnp.float32).at[0, : b.shape[0]].set(b)


@functools.lru_cache(maxsize=None)
def _build_sc_msg():
    mesh = plsc.VectorSubcoreMesh(core_axis_name="c", subcore_axis_name="s")

    @functools.partial(
        pl.kernel,
        mesh=mesh,
        out_type=jax.ShapeDtypeStruct((NC, N, HP), jnp.float32),
        scratch_types=[
            pltpu.VMEM((2, C), jnp.int32),
            pltpu.VMEM((2, C), jnp.int32),
            pltpu.VMEM((2, C, HP), jnp.float32),
            pltpu.VMEM((2, C, HP), jnp.float32),
            pltpu.VMEM((ZR, HP), jnp.float32),
            pltpu.VMEM_SHARED((N, HP), jnp.float32),
            pltpu.SemaphoreType.DMA,
            pltpu.SemaphoreType.DMA,
            pltpu.SemaphoreType.DMA,
            pltpu.SemaphoreType.DMA,
        ],
    )
    def _sc_msg_kernel(h_hbm, elin_hbm, src_hbm, dst_hbm, out_hbm,
                       idx_s, idx_d, elin_v, rows_v, zbuf, acc,
                       sem0, sem1, ssem0, ssem1):
        _sc_msg_body(h_hbm, elin_hbm, src_hbm, dst_hbm, out_hbm,
                     idx_s, idx_d, elin_v, rows_v, zbuf, acc,
                     (sem0, sem1), (ssem0, ssem1))

    return _sc_msg_kernel


def _sc_msg(h, elin, src, dst):
    return _build_sc_msg()(h, elin, src, dst)


def _sc_msg_body(h_hbm, elin_hbm, src_hbm, dst_hbm, out_hbm,
                 idx_s, idx_d, elin_v, rows_v, zbuf, acc, sems, ssems):
    cid = lax.axis_index("c")
    sid = lax.axis_index("s")
    base0 = (cid * NS + sid) * EPW

    def _zrow(r, _):
        for c8 in range(HP // 16):
            zbuf[r, pl.ds(c8 * 16, 16)] = jnp.zeros((16,), jnp.float32)
        return _
    lax.fori_loop(0, ZR, _zrow, None)
    row_base = sid * RA
    nz = jnp.where(sid == NS - 1, (RA + 16) // ZR, RA // ZR)

    def _zcp(j, _):
        pltpu.sync_copy(zbuf, acc.at[pl.ds(row_base + j * ZR, ZR)])
        return _
    lax.fori_loop(0, nz, _zcp, None)
    plsc.subcore_barrier()

    def _issue(k, b):
        @pl.when(k >= 2)
        def _drain():
            pltpu.make_async_copy(rows_v.at[b], acc.at[idx_d.at[b]],
                                  ssems[b]).wait()
        base = base0 + k * C
        pltpu.sync_copy(src_hbm.at[pl.ds(base, C)], idx_s.at[b])
        pltpu.sync_copy(dst_hbm.at[pl.ds(base, C)], idx_d.at[b])
        pltpu.async_copy(elin_hbm.at[pl.ds(base, C)], elin_v.at[b], sems[b])
        pltpu.async_copy(h_hbm.at[idx_s.at[b]], rows_v.at[b], sems[b])

    def _proc(k, b):
        @pl.when(k < NCHUNK - 1)
        def _nxt():
            _issue(k + 1, 1 - b)
        base = base0 + k * C
        pltpu.make_async_copy(elin_hbm.at[pl.ds(base, C)], elin_v.at[b],
                              sems[b]).wait()
        pltpu.make_async_copy(h_hbm.at[idx_s.at[b]], rows_v.at[b],
                              sems[b]).wait()

        def _row(r, __):
            for dr in range(2):
                row = 2 * r + dr
                for c8 in range(HP // 16):
                    sl = pl.ds(c8 * 16, 16)
                    rows_v[b, row, sl] = jnp.maximum(
                        rows_v[b, row, sl] + elin_v[b, row, sl], 0.0)
            return __
        lax.fori_loop(0, C // 2, _row, None)
        pltpu.async_copy(rows_v.at[b], acc.at[idx_d.at[b]], ssems[b],
                         add=True)

    _issue(0, 0)

    def _chunk(k, _):
        @pl.when(k % 2 == 0)
        def _even():
            _proc(k, 0)

        @pl.when(k % 2 == 1)
        def _odd():
            _proc(k, 1)
        return _
    lax.fori_loop(0, NCHUNK, _chunk, None)

    for b in range(2):
        pltpu.make_async_copy(rows_v.at[b], acc.at[idx_d.at[b]],
                              ssems[b]).wait()
    plsc.subcore_barrier()
    for j in range(RA // DR):
        row0 = row_base + j * DR
        pltpu.sync_copy(acc.at[pl.ds(row0, DR)], out_hbm.at[cid, pl.ds(row0, DR)])

    @pl.when(sid == NS - 1)
    def _dtail():
        pltpu.sync_copy(acc.at[pl.ds(row_base + RA, 16)],
                        out_hbm.at[cid, pl.ds(row_base + RA, 16)])


def _pair_pipeline(ts_hbm, td_hbm, src_hbm, dst_hbm, out_hbm,
                   idx_s, idx_d, gs, gd, sems, compute_store):
    cid = lax.axis_index("c")
    sid = lax.axis_index("s")
    base0 = (cid * NS + sid) * EPW

    def _issue(k, b):
        base = base0 + k * C
        pltpu.sync_copy(src_hbm.at[pl.ds(base, C)], idx_s.at[b])
        pltpu.sync_copy(dst_hbm.at[pl.ds(base, C)], idx_d.at[b])
        pltpu.async_copy(ts_hbm.at[idx_s.at[b]], gs.at[b], sems[b])
        pltpu.async_copy(td_hbm.at[idx_d.at[b]], gd.at[b], sems[b])

    def _proc(k, b):
        @pl.when(k < NCHUNK - 1)
        def _nxt():
            _issue(k + 1, 1 - b)
        pltpu.make_async_copy(ts_hbm.at[idx_s.at[b]], gs.at[b], sems[b]).wait()
        pltpu.make_async_copy(td_hbm.at[idx_d.at[b]], gd.at[b], sems[b]).wait()
        compute_store(k, b, base0)

    _issue(0, 0)

    def _chunk(k, _):
        @pl.when(k % 2 == 0)
        def _even():
            _proc(k, 0)

        @pl.when(k % 2 == 1)
        def _odd():
            _proc(k, 1)
        return _
    lax.fori_loop(0, NCHUNK, _chunk, None)


@functools.lru_cache(maxsize=None)
def _build_sc_pair_f32():
    mesh = plsc.VectorSubcoreMesh(core_axis_name="c", subcore_axis_name="s")

    @functools.partial(
        pl.kernel,
        mesh=mesh,
        out_type=jax.ShapeDtypeStruct((E, HP), jnp.float32),
        scratch_types=[
            pltpu.VMEM((2, C), jnp.int32),
            pltpu.VMEM((2, C), jnp.int32),
            pltpu.VMEM((2, C, HP), jnp.float32),
            pltpu.VMEM((2, C, HP), jnp.float32),
            pltpu.SemaphoreType.DMA,
            pltpu.SemaphoreType.DMA,
        ],
    )
    def _sc_pair(ts_hbm, td_hbm, src_hbm, dst_hbm, out_hbm,
                 idx_s, idx_d, gs, gd, sem0, sem1):
        def _cs(k, b, base0):
            def _row(r, __):
                for dr in range(2):
                    row = 2 * r + dr
                    for c8 in range(HP // 16):
                        sl = pl.ds(c8 * 16, 16)
                        gs[b, row, sl] = gs[b, row, sl] + gd[b, row, sl]
                return __
            lax.fori_loop(0, C // 2, _row, None)
            pltpu.sync_copy(gs.at[b], out_hbm.at[pl.ds(base0 + k * C, C)])

        _pair_pipeline(ts_hbm, td_hbm, src_hbm, dst_hbm, out_hbm,
                       idx_s, idx_d, gs, gd, (sem0, sem1), _cs)

    return _sc_pair


@functools.lru_cache(maxsize=None)
def _build_sc_pair_packed():
    mesh = plsc.VectorSubcoreMesh(core_axis_name="c", subcore_axis_name="s")

    @functools.partial(
        pl.kernel,
        mesh=mesh,
        out_type=[jax.ShapeDtypeStruct((E, HP), jnp.int32),
                  jax.ShapeDtypeStruct((E, HP), jnp.int32)],
        scratch_types=[
            pltpu.VMEM((2, C), jnp.int32),
            pltpu.VMEM((2, C), jnp.int32),
            pltpu.VMEM((2, C, HP), jnp.int32),
            pltpu.VMEM((2, C, HP), jnp.int32),
            pltpu.SemaphoreType.DMA,
            pltpu.SemaphoreType.DMA,
        ],
    )
    def _sc_pair(ts_hbm, td_hbm, src_hbm, dst_hbm, outs_hbm, outd_hbm,
                 idx_s, idx_d, gs, gd, sem0, sem1):
        def _cs(k, b, base0):
            base = base0 + k * C
            pltpu.sync_copy(gs.at[b], outs_hbm.at[pl.ds(base, C)])
            pltpu.sync_copy(gd.at[b], outd_hbm.at[pl.ds(base, C)])

        _pair_pipeline(ts_hbm, td_hbm, src_hbm, dst_hbm, None,
                       idx_s, idx_d, gs, gd, (sem0, sem1), _cs)

    return _sc_pair


def _sc_pair_h(ts, td, src, dst):
    return _build_sc_pair_f32()(ts, td, src, dst)


def _sc_pair_w(ts, td, src, dst):
    return _build_sc_pair_packed()(ts, td, src, dst)


_EB = 2000


def _dot(a, b):
    return jnp.dot(a, b, preferred_element_type=jnp.float32)


def _node_emb_body(x_ref, w_ref, b_ref, h_ref):
    h_ref[...] = _dot(x_ref[...], w_ref[...]) + b_ref[...]


def _edge0_body(ea_ref, we_ref, be_ref, gw_ref, gb_ref, e0_ref, el_ref):
    e0 = _dot(ea_ref[...], we_ref[...]) + be_ref[...]
    e0_ref[...] = e0.astype(jnp.bfloat16)
    el_ref[...] = _dot(e0, gw_ref[...]) + gb_ref[...]


def _node_upd0_body(h_ref, ag_ref, w1_ref, b1_ref, w2_ref, b2_ref,
                    gam_ref, bet_ref, w1a_ref, w1b_ref,
                    hn_ref, ts_ref, td_ref):
    h = h_ref[...]
    g = h + ag_ref[0] + ag_ref[1]
    t = jnp.maximum(_dot(g, w1_ref[...]) + b1_ref[...], 0.0)
    nn = _dot(t, w2_ref[...]) + b2_ref[...]
    mu = jnp.mean(nn, axis=0, keepdims=True)
    var = jnp.mean((nn - mu) * (nn - mu), axis=0, keepdims=True)
    bn = (nn - mu) * lax.rsqrt(var + 1e-5) * gam_ref[...] + bet_ref[...]
    hn = (h + jnp.maximum(bn, 0.0)) * 0.5
    hn_ref[...] = hn
    ts_ref[...] = _dot(hn, w1a_ref[...])
    td_ref[...] = _dot(hn, w1b_ref[...])


def _node_upd1_body(h_ref, ag_ref, w1_ref, b1_ref, w2_ref, b2_ref,
                    gam_ref, bet_ref, w1a_ref, w1b_ref, mwa_ref, mwb_ref,
                    hn_ref, ts_ref, td_ref):
    h = h_ref[...]
    g = h + ag_ref[0] + ag_ref[1]
    t = jnp.maximum(_dot(g, w1_ref[...]) + b1_ref[...], 0.0)
    nn = _dot(t, w2_ref[...]) + b2_ref[...]
    mu = jnp.mean(nn, axis=0, keepdims=True)
    var = jnp.mean((nn - mu) * (nn - mu), axis=0, keepdims=True)
    bn = (nn - mu) * lax.rsqrt(var + 1e-5) * gam_ref[...] + bet_ref[...]
    hn = (h + jnp.maximum(bn, 0.0)) * 0.5
    hn_ref[...] = hn
    hr = jnp.maximum(hn, 0.0)
    z = jnp.zeros((hn.shape[0], 32), jnp.int32)
    a = _dot(hn, w1a_ref[...])
    r = _dot(hr, mwa_ref[...])
    ts_ref[...] = jnp.concatenate(
        [_pack_cols(a[:, :64], a[:, 64:]),
         _pack_cols(r[:, :32], r[:, 32:64]), z], axis=1)
    a = _dot(hn, w1b_ref[...])
    r = _dot(hr, mwb_ref[...])
    td_ref[...] = jnp.concatenate(
        [_pack_cols(a[:, :64], a[:, 64:]),
         _pack_cols(r[:, :32], r[:, 32:64]), z], axis=1)


def _edge_upd_body(s_ref, e_ref, w1c_ref, eb1_ref, ew2_ref, eb2_ref,
                   gw_ref, gb_ref, e1_ref, el_ref):
    e = e_ref[...].astype(jnp.float32)
    t = jnp.maximum(s_ref[...] + _dot(e, w1c_ref[...]) + eb1_ref[...], 0.0)
    em = _dot(t, ew2_ref[...]) + eb2_ref[...]
    e1 = e + em * 0.5
    e1_ref[...] = e1.astype(jnp.bfloat16)
    el_ref[...] = _dot(e1, gw_ref[...]) + gb_ref[...]


def _edge_final_body(ss_ref, sd_ref, e_ref, w1c_ref, eb1_ref, ew2_ref, eb2_ref,
                     mwc_ref, mb1_ref, mw2_ref, mb2_ref, mw3_ref, mb3_ref,
                     out_ref):
    e = e_ref[...].astype(jnp.float32)
    ws = ss_ref[...]
    wd = sd_ref[...]
    sa = jnp.concatenate(
        [_unpack_hi(ws[:, :64]) + _unpack_hi(wd[:, :64]),
         _unpack_lo(ws[:, :64]) + _unpack_lo(wd[:, :64])], axis=1)
    sr = jnp.concatenate(
        [_unpack_hi(ws[:, 64:96]) + _unpack_hi(wd[:, 64:96]),
         _unpack_lo(ws[:, 64:96]) + _unpack_lo(wd[:, 64:96])], axis=1)
    t = jnp.maximum(sa + _dot(e, w1c_ref[...]) + eb1_ref[...], 0.0)
    em = _dot(t, ew2_ref[...]) + eb2_ref[...]
    e2 = e + em * 0.5
    z1 = jnp.maximum(sr + _dot(e2, mwc_ref[...]) + mb1_ref[...], 0.0)
    z2 = jnp.maximum(_dot(z1, mw2_ref[...]) + mb2_ref[...], 0.0)
    out_ref[...] = _dot(z2, mw3_ref[...]) + mb3_ref[...]


def _wspec(r, c):
    return pl.BlockSpec((r, c), lambda i: (0, 0))


def _f32(shape):
    return jax.ShapeDtypeStruct(shape, jnp.float32)


def _node_emb(x, wn, bn):
    return pl.pallas_call(_node_emb_body, out_shape=_f32((N, HP)))(x, wn, bn)


def _edge0(ea, we, be, gw, gb):
    grid = E // _EB
    return pl.pallas_call(
        _edge0_body,
        grid=grid,
        in_specs=[pl.BlockSpec((_EB, 16), lambda i: (i, 0)),
                  _wspec(16, HP), _wspec(1, HP), _wspec(HP, HP), _wspec(1, HP)],
        out_specs=[pl.BlockSpec((_EB, HP), lambda i: (i, 0)),
                   pl.BlockSpec((_EB, HP), lambda i: (i, 0))],
        out_shape=[jax.ShapeDtypeStruct((E, HP), jnp.bfloat16),
                   _f32((E, HP))],
    )(ea, we, be, gw, gb)


def _node_upd0(h, ag, w):
    return pl.pallas_call(
        _node_upd0_body,
        out_shape=[_f32((N, HP)), _f32((N, HP)), _f32((N, HP))],
    )(h, ag, w['W1'], w['b1'], w['W2'], w['b2'], w['gam'], w['bet'],
      w['W1a'], w['W1b'])


def _node_upd1(h, ag, w, mwa, mwb):
    return pl.pallas_call(
        _node_upd1_body,
        out_shape=[_f32((N, HP)),
                   jax.ShapeDtypeStruct((N, HP), jnp.int32),
                   jax.ShapeDtypeStruct((N, HP), jnp.int32)],
    )(h, ag, w['W1'], w['b1'], w['W2'], w['b2'], w['gam'], w['bet'],
      w['W1a'], w['W1b'], mwa, mwb)


def _edge_upd(s, e, w, gw, gb):
    grid = E // _EB
    eb = pl.BlockSpec((_EB, HP), lambda i: (i, 0))
    return pl.pallas_call(
        _edge_upd_body,
        grid=grid,
        in_specs=[eb, eb, _wspec(HP, HP), _wspec(1, HP), _wspec(HP, HP),
                  _wspec(1, HP), _wspec(HP, HP), _wspec(1, HP)],
        out_specs=[eb, eb],
        out_shape=[jax.ShapeDtypeStruct((E, HP), jnp.bfloat16),
                   _f32((E, HP))],
    )(s, e, w['W1c'], w['eb1'], w['eW2'], w['eb2'], gw, gb)


def _edge_final(ss, sd, e, w, mwc, mb1, mw2, mb2, mw3, mb3):
    grid = E // _EB
    return pl.pallas_call(
        _edge_final_body,
        grid=grid,
        in_specs=[pl.BlockSpec((_EB, HP), lambda i: (i, 0)),
                  pl.BlockSpec((_EB, HP), lambda i: (i, 0)),
                  pl.BlockSpec((_EB, HP), lambda i: (i, 0)),
                  _wspec(HP, HP), _wspec(1, HP), _wspec(HP, HP), _wspec(1, HP),
                  _wspec(HP, RP), _wspec(1, RP), _wspec(RP, RP), _wspec(1, RP),
                  _wspec(RP, 2), _wspec(1, 2)],
        out_specs=pl.BlockSpec((_EB, 2), lambda i: (i, 0)),
        out_shape=_f32((E, 2)),
    )(ss, sd, e, w['W1c'], w['eb1'], w['eW2'], w['eb2'],
      mwc, mb1, mw2, mb2, mw3, mb3)


def _layer_weights(p):
    return dict(
        gW=_pad2(p['gine_lin_W'], HP, HP), gb=_pad1(p['gine_lin_b'], HP),
        W1=_pad2(p['gmlp_W1'], HP, HP), b1=_pad1(p['gmlp_b1'], HP),
        W2=_pad2(p['gmlp_W2'], HP, HP), b2=_pad1(p['gmlp_b2'], HP),
        gam=_pad1(p['bn_gamma'], HP), bet=_pad1(p['bn_beta'], HP),
        W1a=_pad2(p['emlp_W1'][0:100], HP, HP),
        W1b=_pad2(p['emlp_W1'][100:200], HP, HP),
        W1c=_pad2(p['emlp_W1'][200:300], HP, HP),
        eb1=_pad1(p['emlp_b1'], HP),
        eW2=_pad2(p['emlp_W2'], HP, HP), eb2=_pad1(p['emlp_b2'], HP),
    )


def kernel(x, edge_attr, params, edge_index):
    src = edge_index[0]
    dst = edge_index[1]
    P = params
    w0 = _layer_weights(P['layers'][0])
    w1 = _layer_weights(P['layers'][1])
    wn = _pad2(P['node_emb_W'], 128, HP)
    bn = _pad1(P['node_emb_b'], HP)
    we = _pad2(P['edge_emb_W'], 16, HP)
    be = _pad1(P['edge_emb_b'], HP)
    mwa = _pad2(P['mlp_W1'][0:100], HP, RP)
    mwb = _pad2(P['mlp_W1'][100:200], HP, RP)
    mwc = _pad2(P['mlp_W1'][200:300], HP, RP)
    mb1 = _pad1(P['mlp_b1'], RP)
    mw2 = _pad2(P['mlp_W2'], RP, RP)
    mb2 = _pad1(P['mlp_b2'], RP)
    mw3 = _pad2(P['mlp_W3'], RP, 2)
    mb3 = _pad1(P['mlp_b3'], 2)

    h = _node_emb(x, wn, bn)
    e, elin = _edge0(edge_attr, we, be, w0['gW'], w0['gb'])

    ag = _sc_msg(h, elin, src, dst)
    h, ts, td = _node_upd0(h, ag, w0)
    s = _sc_pair_h(ts, td, src, dst)
    e, elin = _edge_upd(s, e, w0, w1['gW'], w1['gb'])

    ag = _sc_msg(h, elin, src, dst)
    h, ts, td = _node_upd1(h, ag, w1, mwa, mwb)
    ss, sd = _sc_pair_w(ts, td, src, dst)
    out = _edge_final(ss, sd, e, w1, mwc, mb1, mw2, mb2, mw3, mb3)
    return out
